# baseline (speedup 1.0000x reference)
import jax
import jax.numpy as jnp
from jax import lax
from jax.experimental import pallas as pl
from jax.experimental.pallas import tpu as pltpu
from jax.experimental.pallas import tpu_sc as plsc

_B, _T, _H, _W = 1024, 8, 16, 16
_J = _T * _H * _W
_NE = 64
_VOCAB = 10
_NC, _NS, _L = 2, 16, 16
_NW = _NC * _NS
_SPW = _B // _NW
_CW = 64


def _counts_body(x_hbm, out_hbm, xall, hist, rowhist, ostage):
    cid = lax.axis_index("c")
    sid = lax.axis_index("s")
    wid = sid * _NC + cid
    base = wid * _SPW
    pltpu.sync_copy(x_hbm.at[pl.ds(base * _J, _SPW * _J)], xall)
    lane16 = lax.iota(jnp.int32, _L) * _L
    ones = jnp.ones((_L,), jnp.float32)
    zeros16 = jnp.zeros((_L,), jnp.float32)

    def sample_body(s, carry):
        for l in range(_L):
            hist[pl.ds(l * _L, _L)] = zeros16
            rowhist[pl.ds(l * _L, _L)] = zeros16
        colacc = zeros16
        soff = s * _J
        for i in range(_J // _L):
            xv = xall[pl.ds(soff + i * _L, _L)]
            maskf = jnp.minimum(xv, 1).astype(jnp.float32)
            colacc = colacc + maskf
            plsc.addupdate_scatter(hist, [lane16 + xv], ones)
            plsc.addupdate_scatter(rowhist, [lane16 + (i % _H)], maskf)
        valcnt = hist[pl.ds(0, _L)]
        rowcnt = rowhist[pl.ds(0, _L)]
        for l in range(1, _L):
            valcnt = valcnt + hist[pl.ds(l * _L, _L)]
            rowcnt = rowcnt + rowhist[pl.ds(l * _L, _L)]
        ostage[s, pl.ds(0, _L)] = rowcnt
        ostage[s, pl.ds(_L, _L)] = colacc
        ostage[s, pl.ds(2 * _L, _L)] = valcnt
        ostage[s, pl.ds(3 * _L, _L)] = zeros16
        return carry
    lax.fori_loop(0, _SPW, sample_body, 0)
    pltpu.sync_copy(ostage, out_hbm.at[pl.ds(base, _SPW)])


def _sc_counts(x2):
    mesh = plsc.VectorSubcoreMesh(core_axis_name="c", subcore_axis_name="s",
                                  num_cores=_NC, num_subcores=_NS)
    fn = pl.kernel(
        _counts_body,
        out_type=jax.ShapeDtypeStruct((_B, _CW), jnp.float32),
        mesh=mesh,
        compiler_params=pltpu.CompilerParams(needs_layout_passes=False),
        scratch_types=[
            pltpu.VMEM((_SPW * _J,), jnp.int32),
            pltpu.VMEM((_L * _L,), jnp.float32),
            pltpu.VMEM((_L * _L,), jnp.float32),
            pltpu.VMEM((_SPW, _CW), jnp.float32),
        ],
    )
    return fn(x2)


def _combine_body(cnt_ref, hp_ref, row_ref, col_ref, val_ref, w_ref, b_ref,
                  out_ref):
    counts = cnt_ref[...]
    vmask = (lax.broadcasted_iota(jnp.int32, (_VOCAB, 1), 0) != 0
             ).astype(jnp.float32)
    table = jnp.concatenate(
        [row_ref[...], col_ref[...], val_ref[...] * vmask,
         jnp.zeros((_CW - 2 * _H - _VOCAB, _NE), jnp.float32)], axis=0)
    num = jnp.dot(counts, table, preferred_element_type=jnp.float32)
    cnt0 = counts[:, 2 * _L:2 * _L + 1]
    den = jnp.maximum(float(_J) - cnt0, 1.0)
    h = num / den
    dn = (((1,), (1,)), ((), ()))
    out = lax.dot_general(h, w_ref[:, :_NE], dn,
                          preferred_element_type=jnp.float32)
    out = out + lax.dot_general(hp_ref[...], w_ref[:, _NE:], dn,
                                preferred_element_type=jnp.float32)
    out_ref[...] = out + b_ref[...]


@jax.jit
def kernel(x, h_parent, row_embed, col_embed, val_embed, head_w, head_b):
    x2 = x.reshape(_B * _J).astype(jnp.int32)
    counts = _sc_counts(x2)
    nd = head_w.shape[0]
    out = pl.pallas_call(
        _combine_body,
        out_shape=jax.ShapeDtypeStruct((_B, nd), jnp.float32),
    )(counts, h_parent, row_embed, col_embed, val_embed, head_w,
      head_b.reshape(1, -1))
    return out


# speedup vs baseline: 4.8876x; 4.8876x over previous
import functools

import jax
import jax.numpy as jnp
from jax.experimental import pallas as pl
from jax.experimental.pallas import tpu as pltpu

_B, _T, _H, _W = 1024, 8, 16, 16
_J = _T * _H * _W
_NE = 64
_VOCAB = 10
_BB = 128


def _body(x_ref, hp_ref, row_ref, col_ref, val_ref, w_ref, b_ref, out_ref):
    xb = x_ref[...]
    maskf = (xb != 0).astype(jnp.float32)

    j = jax.lax.broadcasted_iota(jnp.int32, (_J, 32), 0)
    k = jax.lax.broadcasted_iota(jnp.int32, (_J, 32), 1)
    rsel = (((j // _W) % _H) == k).astype(jnp.float32)
    csel = ((j % _W) == (k - _H)).astype(jnp.float32)
    ltk = (k < _H).astype(jnp.float32)
    m_pos = rsel * ltk + csel * (1.0 - ltk)
    poscnt = jnp.dot(maskf, m_pos, preferred_element_type=jnp.float32)

    num = jnp.dot(poscnt[:, :_H], row_ref[...],
                  preferred_element_type=jnp.float32)
    num = num + jnp.dot(poscnt[:, _H:], col_ref[...],
                        preferred_element_type=jnp.float32)

    for v in range(1, _VOCAB):
        cnt_v = jnp.sum((xb == v).astype(jnp.float32), axis=1, keepdims=True)
        num = num + cnt_v * val_ref[v, :][None, :]

    den = jnp.maximum(jnp.sum(maskf, axis=1, keepdims=True), 1.0)
    h_matrix = num / den

    dn = (((1,), (1,)), ((), ()))
    out = jax.lax.dot_general(h_matrix, w_ref[:, :_NE], dn,
                              preferred_element_type=jnp.float32)
    out = out + jax.lax.dot_general(hp_ref[...], w_ref[:, _NE:], dn,
                                    preferred_element_type=jnp.float32)
    out_ref[...] = out + b_ref[...]


@jax.jit
def kernel(x, h_parent, row_embed, col_embed, val_embed, head_w, head_b):
    b = x.shape[0]
    x2 = x.reshape(b, _J)
    grid = (b // _BB,)
    full = lambda i: (0, 0)
    out = pl.pallas_call(
        _body,
        grid=grid,
        in_specs=[
            pl.BlockSpec((_BB, _J), lambda i: (i, 0)),
            pl.BlockSpec((_BB, _NE), lambda i: (i, 0)),
            pl.BlockSpec(row_embed.shape, full),
            pl.BlockSpec(col_embed.shape, full),
            pl.BlockSpec(val_embed.shape, full),
            pl.BlockSpec(head_w.shape, full),
            pl.BlockSpec((1, head_w.shape[0]), full),
        ],
        out_specs=pl.BlockSpec((_BB, head_w.shape[0]), lambda i: (i, 0)),
        out_shape=jax.ShapeDtypeStruct((b, head_w.shape[0]), jnp.float32),
    )(x2, h_parent, row_embed, col_embed, val_embed, head_w,
      head_b.reshape(1, -1))
    return out
